# baseline (speedup 1.0000x reference)
import functools

import jax
import jax.numpy as jnp
from jax import lax
from jax.experimental import pallas as pl
from jax.experimental.pallas import tpu as pltpu
from jax.experimental.pallas import tpu_sc as plsc

N = 4096
E = 262144
D = 128
ED = 16
L = 3
H = 8
HD = D // H

NC = 2
NS = 16
NW = NC * NS
EPW = E // NW
G = 128
NG = EPW // G
RPS = N // NS

_f32 = jnp.float32


def _mesh():
    return plsc.VectorSubcoreMesh(core_axis_name="c", subcore_axis_name="s")


def _mesh1():
    return plsc.VectorSubcoreMesh(core_axis_name="c", subcore_axis_name="s",
                                  num_cores=1)


PER = N * ED // D
PDR = N // D


def _prep_sc_body(dst3, ef_p, zerosflat, out, ef_flat, deg_flat, idx_d,
                  packbuf):
    w = lax.axis_index("s")

    pltpu.sync_copy(zerosflat, ef_flat.at[pl.ds(0, RPS * D)])
    pltpu.sync_copy(zerosflat, ef_flat.at[pl.ds(RPS * D, RPS * D)])
    pltpu.sync_copy(zerosflat.at[pl.ds(0, PDR * D)], deg_flat)

    iota16 = lax.iota(jnp.int32, 16)
    ones16 = jnp.full((16,), 1.0, _f32)

    def body(g, carry):
        pltpu.sync_copy(ef_p.at[w, g], packbuf)
        pltpu.sync_copy(dst3.at[w, g], idx_d)
        for k in range(G // 16):
            dvec = idx_d[pl.ds(k * 16, 16)]
            fbase = dvec * ED
            for jj in range(16):
                j = k * 16 + jj
                jsplat = jnp.full((16,), jj, jnp.int32)
                fsplat = fbase.at[jsplat].get(mode="promise_in_bounds")
                ef_row = packbuf[j // 8, pl.ds((j % 8) * ED, ED)]
                plsc.addupdate_scatter(ef_flat, [fsplat + iota16], ef_row)
            plsc.addupdate_scatter(deg_flat, [dvec], ones16)
        return carry

    lax.fori_loop(0, NG, body, 0)
    pltpu.sync_copy(ef_flat, out.at[w, pl.ds(0, PER * D)])
    pltpu.sync_copy(deg_flat, out.at[w, pl.ds(PER * D, PDR * D)])


def _prep_sc(dst3, ef_p, zerosflat):
    return pl.kernel(
        _prep_sc_body,
        out_type=jax.ShapeDtypeStruct((NS, (PER + PDR) * D), _f32),
        mesh=_mesh1(),
        compiler_params=pltpu.CompilerParams(needs_layout_passes=False),
        scratch_types=[
            pltpu.VMEM((PER * D,), _f32),
            pltpu.VMEM((PDR * D,), _f32),
            pltpu.VMEM((G,), jnp.int32),
            pltpu.VMEM((G // 8, D), _f32),
        ],
    )(dst3, ef_p, zerosflat)


NBUF = 4


def _msg_sc_body(nodes, src3, dst3, zeros128, out, acc, idx_s, idx_d, *bs):
    bufs, gsems, ssems = bs[:NBUF], bs[NBUF:2 * NBUF], bs[2 * NBUF:]
    s = lax.axis_index("s")
    w = s

    pltpu.sync_copy(zeros128, acc.at[pl.ds(s * RPS, RPS)])
    pltpu.sync_copy(src3.at[w], idx_s)
    pltpu.sync_copy(dst3.at[w], idx_d)
    plsc.subcore_barrier()

    for p in range(NBUF):
        pltpu.async_copy(nodes.at[idx_s.at[p]], bufs[p], gsems[p])

    def body(q, carry):
        for p in range(NBUF):
            g = NBUF * q + p
            pltpu.make_async_copy(nodes.at[idx_s.at[g]], bufs[p], gsems[p]).wait()
            pltpu.async_copy(bufs[p], acc.at[idx_d.at[g]], ssems[p], add=True)

            @pl.when(g + NBUF < NG)
            def _():
                pltpu.make_async_copy(bufs[p], acc.at[idx_d.at[g]],
                                      ssems[p]).wait()
                pltpu.async_copy(nodes.at[idx_s.at[g + NBUF]], bufs[p],
                                 gsems[p])

        return carry

    lax.fori_loop(0, NG // NBUF, body, 0)
    for p in range(NBUF):
        pltpu.make_async_copy(bufs[p], acc.at[idx_d.at[NG - NBUF + p]],
                              ssems[p]).wait()
    plsc.subcore_barrier()
    pltpu.sync_copy(acc.at[pl.ds(s * RPS, RPS)], out.at[pl.ds(s * RPS, RPS)])


def _msg_sc(nodes, src3, dst3, zeros128):
    return pl.kernel(
        _msg_sc_body,
        out_type=jax.ShapeDtypeStruct((N, D), _f32),
        mesh=_mesh1(),
        scratch_types=[
            pltpu.VMEM_SHARED((N, D), _f32),
            pltpu.VMEM((NG, G), jnp.int32),
            pltpu.VMEM((NG, G), jnp.int32),
        ]
        + [pltpu.VMEM((G, D), _f32)] * NBUF
        + [pltpu.SemaphoreType.DMA] * (2 * NBUF),
    )(nodes, src3, dst3, zeros128)


def _dotT(x, w):
    return lax.dot_general(x, w, (((1,), (1,)), ((), ())),
                           preferred_element_type=_f32)


BR = 512


def _proj_body(x_ref, w_ref, b_ref, o_ref):
    o_ref[...] = _dotT(x_ref[...], w_ref[...]) + b_ref[...]


def _proj(x, w, b, out_cols):
    n = x.shape[0]
    return pl.pallas_call(
        _proj_body,
        grid=(n // BR,),
        in_specs=[
            pl.BlockSpec((BR, x.shape[1]), lambda i: (i, 0)),
            pl.BlockSpec(w.shape, lambda i: (0, 0)),
            pl.BlockSpec((1, out_cols), lambda i: (0, 0)),
        ],
        out_specs=pl.BlockSpec((BR, out_cols), lambda i: (i, 0)),
        out_shape=jax.ShapeDtypeStruct((n, out_cols), _f32),
    )(x, w, b.reshape(1, -1))


def _prep2_body(p_ref, wbd_ref, s_ref, ep_ref):
    s = jnp.sum(p_ref[...], axis=0)
    s_ref[...] = s
    ep_ref[...] = lax.dot_general(s[0:PER], wbd_ref[...],
                                  (((1,), (0,)), ((), ())),
                                  preferred_element_type=_f32)


def _prep2(efp32, wbd):
    return pl.pallas_call(
        _prep2_body,
        in_specs=[
            pl.BlockSpec((NW, PER + PDR, D), lambda: (0, 0, 0)),
            pl.BlockSpec((D, 8 * D), lambda: (0, 0)),
        ],
        out_specs=[
            pl.BlockSpec((PER + PDR, D), lambda: (0, 0)),
            pl.BlockSpec((PER, 8 * D), lambda: (0, 0)),
        ],
        out_shape=[
            jax.ShapeDtypeStruct((PER + PDR, D), _f32),
            jax.ShapeDtypeStruct((PER, 8 * D), _f32),
        ],
    )(efp32, wbd)


def _layer_body(nodes_ref, m0_ref, m1_ref, degb_ref, ep_ref, bep_ref,
                w1a_ref, w1b_ref, b1_ref, g1_ref, bt1_ref, w2_ref, b2_ref,
                o_ref):
    nodes = nodes_ref[...]
    agg = (m0_ref[...] + m1_ref[...]
           + degb_ref[...] * (nodes + bep_ref[...]) + ep_ref[...])
    h = _dotT(nodes, w1a_ref[...]) + _dotT(agg, w1b_ref[...]) + b1_ref[...]
    mu = jnp.mean(h, axis=1, keepdims=True)
    dvar = h - mu
    var = jnp.mean(dvar * dvar, axis=1, keepdims=True)
    hn = dvar * lax.rsqrt(var + 1e-5) * g1_ref[...] + bt1_ref[...]
    r = jnp.maximum(hn, 0.0)
    o_ref[...] = _dotT(r, w2_ref[...]) + b2_ref[...] + nodes


def _layer(nodes, m0, m1, degb, ep, bep, w1a, w1b, b1, g1, bt1, w2, b2):
    row = lambda i: (i, 0)
    full = lambda i: (0, 0)
    return pl.pallas_call(
        _layer_body,
        grid=(N // BR,),
        in_specs=[pl.BlockSpec((BR, D), row)] * 5
        + [pl.BlockSpec((1, D), full)]
        + [pl.BlockSpec((D, D), full), pl.BlockSpec((D, D), full)]
        + [pl.BlockSpec((1, D), full)] * 3
        + [pl.BlockSpec((D, D), full), pl.BlockSpec((1, D), full)],
        out_specs=pl.BlockSpec((BR, D), row),
        out_shape=jax.ShapeDtypeStruct((N, D), _f32),
    )(nodes, m0, m1, degb, ep, bep.reshape(1, D), w1a, w1b, b1.reshape(1, D),
      g1.reshape(1, D), bt1.reshape(1, D), w2, b2.reshape(1, D))


QB = 256


def _attn_body(q_ref, k_ref, v_ref, wo_ref, bo_ref, nodes_ref, o_ref, acc):
    h = pl.program_id(1)
    lane = lax.broadcasted_iota(jnp.int32, (1, D), 1)
    msk = (lane // HD == h).astype(_f32)
    km = k_ref[...] * msk
    s = _dotT(q_ref[...], km) * (1.0 / (HD ** 0.5))
    m = jnp.max(s, axis=1, keepdims=True)
    p = jnp.exp(s - m)
    z = jnp.sum(p, axis=1, keepdims=True)
    vm = v_ref[...] * msk
    ctx = lax.dot_general(p, vm, (((1,), (0,)), ((), ())),
                          preferred_element_type=_f32) / z

    @pl.when(h == 0)
    def _():
        acc[...] = ctx

    @pl.when(h > 0)
    def _():
        acc[...] += ctx

    @pl.when(h == H - 1)
    def _():
        o_ref[...] = _dotT(acc[...], wo_ref[...]) + bo_ref[...] + nodes_ref[...]


def _attn(q, k, v, w_out, b_out, nodes):
    qrow = lambda i, h: (i, 0)
    full = lambda i, h: (0, 0)
    return pl.pallas_call(
        _attn_body,
        grid=(N // QB, H),
        in_specs=[
            pl.BlockSpec((QB, D), qrow),
            pl.BlockSpec((N, D), full),
            pl.BlockSpec((N, D), full),
            pl.BlockSpec((D, D), full),
            pl.BlockSpec((1, D), full),
            pl.BlockSpec((QB, D), qrow),
        ],
        out_specs=pl.BlockSpec((QB, D), qrow),
        out_shape=jax.ShapeDtypeStruct((N, D), _f32),
        scratch_shapes=[pltpu.VMEM((QB, D), _f32)],
        compiler_params=pltpu.CompilerParams(
            dimension_semantics=("parallel", "arbitrary")),
    )(q, k, v, w_out, b_out.reshape(1, D), nodes)


def kernel(node_features, edge_index, edge_features, W_np, b_np, W_ep, b_ep,
           W1, b1, g1, bt1, W2, b2, W_in, b_in, W_out, b_out):
    src3 = edge_index[0].reshape(NW, NG, G)
    dst3 = edge_index[1].reshape(NW, NG, G)
    ef_p = edge_features.reshape(NW, NG, G // 8, D)
    zeros128 = jnp.zeros((RPS, D), _f32)
    zerosflat = jnp.zeros((RPS * D,), _f32)

    efp_a = _prep_sc(dst3[:NS], ef_p[:NS], zerosflat)
    efp_b = _prep_sc(dst3[NS:], ef_p[NS:], zerosflat)
    efp32 = jnp.concatenate([efp_a, efp_b], axis=0).reshape(NW, PER + PDR, D)

    wbd = jnp.kron(jnp.eye(8, dtype=_f32), W_ep.T)
    ssum, ep_packed = _prep2(efp32, wbd)
    ep = ep_packed.reshape(N, D)
    degb = jnp.broadcast_to(ssum[PER:].reshape(N)[:, None], (N, D))

    nodes = _proj(node_features, W_np, b_np, D)

    for l in range(L):
        part_a = _msg_sc(nodes, src3[:NS], dst3[:NS], zeros128)
        part_b = _msg_sc(nodes, src3[NS:], dst3[NS:], zeros128)
        nodes = _layer(nodes, part_a, part_b, degb, ep, b_ep,
                       W1[l, :, :D], W1[l, :, D:], b1[l], g1[l], bt1[l],
                       W2[l], b2[l])

    qkv = _proj(nodes, W_in, b_in, 3 * D)
    return _attn(qkv[:, :D], qkv[:, D:2 * D], qkv[:, 2 * D:], W_out, b_out,
                 nodes)


# speedup vs baseline: 1.3121x; 1.3121x over previous
import functools

import jax
import jax.numpy as jnp
from jax import lax
from jax.experimental import pallas as pl
from jax.experimental.pallas import tpu as pltpu
from jax.experimental.pallas import tpu_sc as plsc

N = 4096
E = 262144
D = 128
ED = 16
L = 3
H = 8
HD = D // H

NC = 2
NS = 16
NW = NC * NS
EPW = E // NW
G = 128
NG = EPW // G
RPS = N // NS

_f32 = jnp.float32


def _mesh():
    return plsc.VectorSubcoreMesh(core_axis_name="c", subcore_axis_name="s")


def _mesh1():
    return plsc.VectorSubcoreMesh(core_axis_name="c", subcore_axis_name="s",
                                  num_cores=1)


PER = N * ED // D
PDR = N // D


def _prep_sc_body(dst3, ef_p, zerosflat, out, ef_flat, deg_flat, idx_d,
                  packbuf):
    c = lax.axis_index("c")
    s = lax.axis_index("s")
    w = c * NS + s

    pltpu.sync_copy(zerosflat, ef_flat.at[pl.ds(0, RPS * D)])
    pltpu.sync_copy(zerosflat, ef_flat.at[pl.ds(RPS * D, RPS * D)])
    pltpu.sync_copy(zerosflat.at[pl.ds(0, PDR * D)], deg_flat)

    iota16 = lax.iota(jnp.int32, 16)
    ones16 = jnp.full((16,), 1.0, _f32)

    def body(g, carry):
        pltpu.sync_copy(ef_p.at[w, g], packbuf)
        pltpu.sync_copy(dst3.at[w, g], idx_d)
        for k in range(G // 16):
            dvec = idx_d[pl.ds(k * 16, 16)]
            fbase = dvec * ED
            for jj in range(16):
                j = k * 16 + jj
                jsplat = jnp.full((16,), jj, jnp.int32)
                fsplat = fbase.at[jsplat].get(mode="promise_in_bounds")
                ef_row = packbuf[j // 8, pl.ds((j % 8) * ED, ED)]
                plsc.addupdate_scatter(ef_flat, [fsplat + iota16], ef_row)
            plsc.addupdate_scatter(deg_flat, [dvec], ones16)
        return carry

    lax.fori_loop(0, NG, body, 0)
    pltpu.sync_copy(ef_flat, out.at[w, pl.ds(0, PER * D)])
    pltpu.sync_copy(deg_flat, out.at[w, pl.ds(PER * D, PDR * D)])


def _prep_sc(dst3, ef_p, zerosflat):
    return pl.kernel(
        _prep_sc_body,
        out_type=jax.ShapeDtypeStruct((NW, (PER + PDR) * D), _f32),
        mesh=_mesh(),
        compiler_params=pltpu.CompilerParams(needs_layout_passes=False),
        scratch_types=[
            pltpu.VMEM((PER * D,), _f32),
            pltpu.VMEM((PDR * D,), _f32),
            pltpu.VMEM((G,), jnp.int32),
            pltpu.VMEM((G // 8, D), _f32),
        ],
    )(dst3, ef_p, zerosflat)


NBUF = 4


def _msg_sc_body(nodes, src3, dst3, zeros128, out, acc, idx_s, idx_d, *bs):
    bufs, gsems, ssems = bs[:NBUF], bs[NBUF:2 * NBUF], bs[2 * NBUF:]
    c = lax.axis_index("c")
    s = lax.axis_index("s")
    w = c * NS + s

    pltpu.sync_copy(zeros128, acc.at[pl.ds(s * RPS, RPS)])
    pltpu.sync_copy(src3.at[w], idx_s)
    pltpu.sync_copy(dst3.at[w], idx_d)
    plsc.subcore_barrier()

    for p in range(NBUF):
        pltpu.async_copy(nodes.at[idx_s.at[p]], bufs[p], gsems[p])

    def body(q, carry):
        for p in range(NBUF):
            g = NBUF * q + p
            pltpu.make_async_copy(nodes.at[idx_s.at[g]], bufs[p], gsems[p]).wait()
            pltpu.async_copy(bufs[p], acc.at[idx_d.at[g]], ssems[p], add=True)

            @pl.when(g + NBUF < NG)
            def _():
                pltpu.make_async_copy(bufs[p], acc.at[idx_d.at[g]],
                                      ssems[p]).wait()
                pltpu.async_copy(nodes.at[idx_s.at[g + NBUF]], bufs[p],
                                 gsems[p])

        return carry

    lax.fori_loop(0, NG // NBUF, body, 0)
    for p in range(NBUF):
        pltpu.make_async_copy(bufs[p], acc.at[idx_d.at[NG - NBUF + p]],
                              ssems[p]).wait()
    plsc.subcore_barrier()
    pltpu.sync_copy(acc.at[pl.ds(s * RPS, RPS)], out.at[c, pl.ds(s * RPS, RPS)])


def _msg_sc(nodes, src3, dst3, zeros128):
    return pl.kernel(
        _msg_sc_body,
        out_type=jax.ShapeDtypeStruct((NC, N, D), _f32),
        mesh=_mesh(),
        scratch_types=[
            pltpu.VMEM_SHARED((N, D), _f32),
            pltpu.VMEM((NG, G), jnp.int32),
            pltpu.VMEM((NG, G), jnp.int32),
        ]
        + [pltpu.VMEM((G, D), _f32)] * NBUF
        + [pltpu.SemaphoreType.DMA] * (2 * NBUF),
    )(nodes, src3, dst3, zeros128)


MB = 2
G2 = 64
NG2 = EPW // G2


def _msg_prep_sc_body(nodes, src3, dst3, ef_p, zeros128, zerosflat,
                      out, out_efp, acc, ef_flat, deg_flat, *bs):
    bufs = bs[:MB]
    pbufs = bs[MB:2 * MB]
    sbufs = bs[2 * MB:3 * MB]
    dbufs = bs[3 * MB:4 * MB]
    gsems, ssems, esems, isems = (bs[4 * MB:5 * MB], bs[5 * MB:6 * MB],
                                  bs[6 * MB:7 * MB], bs[7 * MB:])
    c = lax.axis_index("c")
    s = lax.axis_index("s")
    w = c * NS + s

    pltpu.sync_copy(zeros128, acc.at[pl.ds(s * RPS, RPS)])
    pltpu.sync_copy(zerosflat, ef_flat.at[pl.ds(0, RPS * D)])
    pltpu.sync_copy(zerosflat, ef_flat.at[pl.ds(RPS * D, RPS * D)])
    pltpu.sync_copy(zerosflat.at[pl.ds(0, PDR * D)], deg_flat)
    plsc.subcore_barrier()

    for p in range(MB):
        pltpu.sync_copy(src3.at[w, p], sbufs[p])
        pltpu.sync_copy(dst3.at[w, p], dbufs[p])
        pltpu.async_copy(nodes.at[sbufs[p].at[0]], bufs[p], gsems[p])
        pltpu.async_copy(ef_p.at[w, p], pbufs[p], esems[p])

    iota16 = lax.iota(jnp.int32, 16)
    ones16 = jnp.full((16,), 1.0, _f32)

    def body(q, carry):
        for p in range(MB):
            g = MB * q + p
            pltpu.make_async_copy(nodes.at[sbufs[p].at[0]], bufs[p],
                                  gsems[p]).wait()
            pltpu.async_copy(bufs[p], acc.at[dbufs[p].at[0]], ssems[p],
                             add=True)
            pltpu.make_async_copy(ef_p.at[w, g], pbufs[p], esems[p]).wait()
            for k in range(G2 // 16):
                dvec = dbufs[p][0, pl.ds(k * 16, 16)]
                fbase = dvec * ED
                for jj in range(16):
                    j = k * 16 + jj
                    jsplat = jnp.full((16,), jj, jnp.int32)
                    fsplat = fbase.at[jsplat].get(mode="promise_in_bounds")
                    ef_row = pbufs[p][j // 8, pl.ds((j % 8) * ED, ED)]
                    plsc.addupdate_scatter(ef_flat, [fsplat + iota16], ef_row)
                plsc.addupdate_scatter(deg_flat, [dvec], ones16)

            @pl.when(g + MB < NG2)
            def _():
                pltpu.make_async_copy(bufs[p], acc.at[dbufs[p].at[0]],
                                      ssems[p]).wait()
                pltpu.sync_copy(src3.at[w, g + MB], sbufs[p])
                pltpu.sync_copy(dst3.at[w, g + MB], dbufs[p])
                pltpu.async_copy(nodes.at[sbufs[p].at[0]], bufs[p], gsems[p])
                pltpu.async_copy(ef_p.at[w, g + MB], pbufs[p], esems[p])

        return carry

    lax.fori_loop(0, NG2 // MB, body, 0)
    for p in range(MB):
        pltpu.make_async_copy(bufs[p], acc.at[dbufs[p].at[0]], ssems[p]).wait()
    pltpu.sync_copy(ef_flat, out_efp.at[w, pl.ds(0, PER * D)])
    pltpu.sync_copy(deg_flat, out_efp.at[w, pl.ds(PER * D, PDR * D)])
    plsc.subcore_barrier()
    pltpu.sync_copy(acc.at[pl.ds(s * RPS, RPS)], out.at[c, pl.ds(s * RPS, RPS)])


def _msg_prep_sc(nodes, src3, dst3, ef_p, zeros128, zerosflat):
    return pl.kernel(
        _msg_prep_sc_body,
        out_type=[
            jax.ShapeDtypeStruct((NC, N, D), _f32),
            jax.ShapeDtypeStruct((NW, (PER + PDR) * D), _f32),
        ],
        mesh=_mesh(),
        compiler_params=pltpu.CompilerParams(needs_layout_passes=False),
        scratch_types=[
            pltpu.VMEM_SHARED((N, D), _f32),
            pltpu.VMEM((PER * D,), _f32),
            pltpu.VMEM((PDR * D,), _f32),
        ]
        + [pltpu.VMEM((G2, D), _f32)] * MB
        + [pltpu.VMEM((G2 // 8, D), _f32)] * MB
        + [pltpu.VMEM((1, G2), jnp.int32)] * MB
        + [pltpu.VMEM((1, G2), jnp.int32)] * MB
        + [pltpu.SemaphoreType.DMA] * (4 * MB),
    )(nodes, src3, dst3, ef_p, zeros128, zerosflat)


def _dotT(x, w):
    return lax.dot_general(x, w, (((1,), (1,)), ((), ())),
                           preferred_element_type=_f32)


BR = 512


def _proj_body(x_ref, w_ref, b_ref, o_ref):
    o_ref[...] = _dotT(x_ref[...], w_ref[...]) + b_ref[...]


def _proj(x, w, b, out_cols):
    n = x.shape[0]
    return pl.pallas_call(
        _proj_body,
        grid=(n // BR,),
        in_specs=[
            pl.BlockSpec((BR, x.shape[1]), lambda i: (i, 0)),
            pl.BlockSpec(w.shape, lambda i: (0, 0)),
            pl.BlockSpec((1, out_cols), lambda i: (0, 0)),
        ],
        out_specs=pl.BlockSpec((BR, out_cols), lambda i: (i, 0)),
        out_shape=jax.ShapeDtypeStruct((n, out_cols), _f32),
    )(x, w, b.reshape(1, -1))


def _prep2_body(p_ref, wbd_ref, s_ref, ep_ref):
    s = jnp.sum(p_ref[...], axis=0)
    s_ref[...] = s
    ep_ref[...] = lax.dot_general(s[0:PER], wbd_ref[...],
                                  (((1,), (0,)), ((), ())),
                                  preferred_element_type=_f32)


def _prep2(efp32, wbd):
    return pl.pallas_call(
        _prep2_body,
        in_specs=[
            pl.BlockSpec((NW, PER + PDR, D), lambda: (0, 0, 0)),
            pl.BlockSpec((D, 8 * D), lambda: (0, 0)),
        ],
        out_specs=[
            pl.BlockSpec((PER + PDR, D), lambda: (0, 0)),
            pl.BlockSpec((PER, 8 * D), lambda: (0, 0)),
        ],
        out_shape=[
            jax.ShapeDtypeStruct((PER + PDR, D), _f32),
            jax.ShapeDtypeStruct((PER, 8 * D), _f32),
        ],
    )(efp32, wbd)


def _layer_body(nodes_ref, m0_ref, m1_ref, degb_ref, ep_ref, bep_ref,
                w1a_ref, w1b_ref, b1_ref, g1_ref, bt1_ref, w2_ref, b2_ref,
                o_ref):
    nodes = nodes_ref[...]
    agg = (m0_ref[...] + m1_ref[...]
           + degb_ref[...] * (nodes + bep_ref[...]) + ep_ref[...])
    h = _dotT(nodes, w1a_ref[...]) + _dotT(agg, w1b_ref[...]) + b1_ref[...]
    mu = jnp.mean(h, axis=1, keepdims=True)
    dvar = h - mu
    var = jnp.mean(dvar * dvar, axis=1, keepdims=True)
    hn = dvar * lax.rsqrt(var + 1e-5) * g1_ref[...] + bt1_ref[...]
    r = jnp.maximum(hn, 0.0)
    o_ref[...] = _dotT(r, w2_ref[...]) + b2_ref[...] + nodes


def _layer(nodes, m0, m1, degb, ep, bep, w1a, w1b, b1, g1, bt1, w2, b2):
    row = lambda i: (i, 0)
    full = lambda i: (0, 0)
    return pl.pallas_call(
        _layer_body,
        grid=(N // BR,),
        in_specs=[pl.BlockSpec((BR, D), row)] * 5
        + [pl.BlockSpec((1, D), full)]
        + [pl.BlockSpec((D, D), full), pl.BlockSpec((D, D), full)]
        + [pl.BlockSpec((1, D), full)] * 3
        + [pl.BlockSpec((D, D), full), pl.BlockSpec((1, D), full)],
        out_specs=pl.BlockSpec((BR, D), row),
        out_shape=jax.ShapeDtypeStruct((N, D), _f32),
    )(nodes, m0, m1, degb, ep, bep.reshape(1, D), w1a, w1b, b1.reshape(1, D),
      g1.reshape(1, D), bt1.reshape(1, D), w2, b2.reshape(1, D))


QB = 256


def _attn_body(q_ref, k_ref, v_ref, wo_ref, bo_ref, nodes_ref, o_ref, acc):
    h = pl.program_id(1)
    lane = lax.broadcasted_iota(jnp.int32, (1, D), 1)
    msk = (lane // HD == h).astype(_f32)
    km = k_ref[...] * msk
    s = _dotT(q_ref[...], km) * (1.0 / (HD ** 0.5))
    m = jnp.max(s, axis=1, keepdims=True)
    p = jnp.exp(s - m)
    z = jnp.sum(p, axis=1, keepdims=True)
    vm = v_ref[...] * msk
    ctx = lax.dot_general(p, vm, (((1,), (0,)), ((), ())),
                          preferred_element_type=_f32) / z

    @pl.when(h == 0)
    def _():
        acc[...] = ctx

    @pl.when(h > 0)
    def _():
        acc[...] += ctx

    @pl.when(h == H - 1)
    def _():
        o_ref[...] = _dotT(acc[...], wo_ref[...]) + bo_ref[...] + nodes_ref[...]


def _attn(q, k, v, w_out, b_out, nodes):
    qrow = lambda i, h: (i, 0)
    full = lambda i, h: (0, 0)
    return pl.pallas_call(
        _attn_body,
        grid=(N // QB, H),
        in_specs=[
            pl.BlockSpec((QB, D), qrow),
            pl.BlockSpec((N, D), full),
            pl.BlockSpec((N, D), full),
            pl.BlockSpec((D, D), full),
            pl.BlockSpec((1, D), full),
            pl.BlockSpec((QB, D), qrow),
        ],
        out_specs=pl.BlockSpec((QB, D), qrow),
        out_shape=jax.ShapeDtypeStruct((N, D), _f32),
        scratch_shapes=[pltpu.VMEM((QB, D), _f32)],
        compiler_params=pltpu.CompilerParams(
            dimension_semantics=("parallel", "arbitrary")),
    )(q, k, v, w_out, b_out.reshape(1, D), nodes)


def kernel(node_features, edge_index, edge_features, W_np, b_np, W_ep, b_ep,
           W1, b1, g1, bt1, W2, b2, W_in, b_in, W_out, b_out):
    src3 = edge_index[0].reshape(NW, NG, G)
    dst3 = edge_index[1].reshape(NW, NG, G)
    ef_p = edge_features.reshape(NW, NG, G // 8, D)
    zeros128 = jnp.zeros((RPS, D), _f32)
    zerosflat = jnp.zeros((RPS * D,), _f32)

    nodes = _proj(node_features, W_np, b_np, D)

    src3m = edge_index[0].reshape(NW, NG2, 1, G2)
    dst3m = edge_index[1].reshape(NW, NG2, 1, G2)
    ef_pm = edge_features.reshape(NW, NG2, G2 // 8, D)
    parts, efp = _msg_prep_sc(nodes, src3m, dst3m, ef_pm, zeros128, zerosflat)
    efp32 = efp.reshape(NW, PER + PDR, D)

    wbd = jnp.kron(jnp.eye(8, dtype=_f32), W_ep.T)
    ssum, ep_packed = _prep2(efp32, wbd)
    ep = ep_packed.reshape(N, D)
    degb = jnp.broadcast_to(ssum[PER:].reshape(N)[:, None], (N, D))

    for l in range(L):
        if l > 0:
            parts = _msg_sc(nodes, src3, dst3, zeros128)
        nodes = _layer(nodes, parts[0], parts[1], degb, ep, b_ep,
                       W1[l, :, :D], W1[l, :, D:], b1[l], g1[l], bt1[l],
                       W2[l], b2[l])

    qkv = _proj(nodes, W_in, b_in, 3 * D)
    return _attn(qkv[:, :D], qkv[:, D:2 * D], qkv[:, 2 * D:], W_out, b_out,
                 nodes)


# speedup vs baseline: 1.4294x; 1.0894x over previous
import functools

import jax
import jax.numpy as jnp
from jax import lax
from jax.experimental import pallas as pl
from jax.experimental.pallas import tpu as pltpu
from jax.experimental.pallas import tpu_sc as plsc

N = 4096
E = 262144
D = 128
ED = 16
L = 3
H = 8
HD = D // H

NC = 2
NS = 16
NW = NC * NS
EPW = E // NW
G = 128
NG = EPW // G
RPS = N // NS

_f32 = jnp.float32


def _mesh():
    return plsc.VectorSubcoreMesh(core_axis_name="c", subcore_axis_name="s")


def _prep_sc_body(dst3, ef_p, zeros128, wide_init, out, acc, idx_d, packbuf,
                  wide):
    c = lax.axis_index("c")
    s = lax.axis_index("s")
    w = c * NS + s

    pltpu.sync_copy(zeros128, acc.at[pl.ds(s * RPS, RPS)])
    pltpu.sync_copy(wide_init, wide)
    pltpu.sync_copy(dst3.at[w], idx_d)
    plsc.subcore_barrier()

    def body(g, carry):
        pltpu.sync_copy(ef_p.at[w, g], packbuf)
        for j in range(G):
            wide[j, pl.ds(0, ED)] = packbuf[j // 8, pl.ds((j % 8) * ED, ED)]
        pltpu.sync_copy(wide, acc.at[idx_d.at[g]], add=True)
        return carry

    lax.fori_loop(0, NG, body, 0)
    plsc.subcore_barrier()
    pltpu.sync_copy(acc.at[pl.ds(s * RPS, RPS)], out.at[c, pl.ds(s * RPS, RPS)])


def _prep_sc(dst3, ef_p, zeros128, wide_init):
    return pl.kernel(
        _prep_sc_body,
        out_type=jax.ShapeDtypeStruct((NC, N, D), _f32),
        mesh=_mesh(),
        scratch_types=[
            pltpu.VMEM_SHARED((N, D), _f32),
            pltpu.VMEM((NG, G), jnp.int32),
            pltpu.VMEM((G // 8, D), _f32),
            pltpu.VMEM((G, D), _f32),
        ],
    )(dst3, ef_p, zeros128, wide_init)


NBUF = 4


def _msg_sc_body(nodes, src3, dst3, zeros128, out, acc, idx_s, idx_d, *bs):
    bufs, gsems, ssems = bs[:NBUF], bs[NBUF:2 * NBUF], bs[2 * NBUF:]
    c = lax.axis_index("c")
    s = lax.axis_index("s")
    w = c * NS + s

    pltpu.sync_copy(zeros128, acc.at[pl.ds(s * RPS, RPS)])
    pltpu.sync_copy(src3.at[w], idx_s)
    pltpu.sync_copy(dst3.at[w], idx_d)
    plsc.subcore_barrier()

    for p in range(NBUF):
        pltpu.async_copy(nodes.at[idx_s.at[p]], bufs[p], gsems[p])

    def body(q, carry):
        for p in range(NBUF):
            g = NBUF * q + p
            pltpu.make_async_copy(nodes.at[idx_s.at[g]], bufs[p], gsems[p]).wait()
            pltpu.async_copy(bufs[p], acc.at[idx_d.at[g]], ssems[p], add=True)

            @pl.when(g + NBUF < NG)
            def _():
                pltpu.make_async_copy(bufs[p], acc.at[idx_d.at[g]],
                                      ssems[p]).wait()
                pltpu.async_copy(nodes.at[idx_s.at[g + NBUF]], bufs[p],
                                 gsems[p])

        return carry

    lax.fori_loop(0, NG // NBUF, body, 0)
    for p in range(NBUF):
        pltpu.make_async_copy(bufs[p], acc.at[idx_d.at[NG - NBUF + p]],
                              ssems[p]).wait()
    plsc.subcore_barrier()
    pltpu.sync_copy(acc.at[pl.ds(s * RPS, RPS)], out.at[c, pl.ds(s * RPS, RPS)])


def _msg_sc(nodes, src3, dst3, zeros128):
    return pl.kernel(
        _msg_sc_body,
        out_type=jax.ShapeDtypeStruct((NC, N, D), _f32),
        mesh=_mesh(),
        scratch_types=[
            pltpu.VMEM_SHARED((N, D), _f32),
            pltpu.VMEM((NG, G), jnp.int32),
            pltpu.VMEM((NG, G), jnp.int32),
        ]
        + [pltpu.VMEM((G, D), _f32)] * NBUF
        + [pltpu.SemaphoreType.DMA] * (2 * NBUF),
    )(nodes, src3, dst3, zeros128)


def _dotT(x, w):
    return lax.dot_general(x, w, (((1,), (1,)), ((), ())),
                           preferred_element_type=_f32)


BR = 512


def _proj_body(x_ref, w_ref, b_ref, o_ref):
    o_ref[...] = _dotT(x_ref[...], w_ref[...]) + b_ref[...]


def _proj(x, w, b, out_cols):
    n = x.shape[0]
    return pl.pallas_call(
        _proj_body,
        grid=(n // BR,),
        in_specs=[
            pl.BlockSpec((BR, x.shape[1]), lambda i: (i, 0)),
            pl.BlockSpec(w.shape, lambda i: (0, 0)),
            pl.BlockSpec((1, out_cols), lambda i: (0, 0)),
        ],
        out_specs=pl.BlockSpec((BR, out_cols), lambda i: (i, 0)),
        out_shape=jax.ShapeDtypeStruct((n, out_cols), _f32),
    )(x, w, b.reshape(1, -1))


def _prep2_body(e0_ref, e1_ref, wpad_ref, cdeg_ref, ep_ref, degb_ref):
    sum_ef = e0_ref[...] + e1_ref[...]
    ep_ref[...] = _dotT(sum_ef, wpad_ref[...])
    degb_ref[...] = _dotT(sum_ef, cdeg_ref[...])


def _prep2(efp, wpad, cdeg):
    return pl.pallas_call(
        _prep2_body,
        grid=(N // BR,),
        in_specs=[
            pl.BlockSpec((BR, D), lambda i: (i, 0)),
            pl.BlockSpec((BR, D), lambda i: (i, 0)),
            pl.BlockSpec((D, D), lambda i: (0, 0)),
            pl.BlockSpec((D, D), lambda i: (0, 0)),
        ],
        out_specs=[
            pl.BlockSpec((BR, D), lambda i: (i, 0)),
            pl.BlockSpec((BR, D), lambda i: (i, 0)),
        ],
        out_shape=[
            jax.ShapeDtypeStruct((N, D), _f32),
            jax.ShapeDtypeStruct((N, D), _f32),
        ],
    )(efp[0], efp[1], wpad, cdeg)


def _layer_body(nodes_ref, m0_ref, m1_ref, degb_ref, ep_ref,
                w1a_ref, w1b_ref, b1_ref, g1_ref, bt1_ref, w2_ref, b2_ref,
                o_ref):
    nodes = nodes_ref[...]
    agg = m0_ref[...] + m1_ref[...] + degb_ref[...] * nodes + ep_ref[...]
    h = _dotT(nodes, w1a_ref[...]) + _dotT(agg, w1b_ref[...]) + b1_ref[...]
    mu = jnp.mean(h, axis=1, keepdims=True)
    dvar = h - mu
    var = jnp.mean(dvar * dvar, axis=1, keepdims=True)
    hn = dvar * lax.rsqrt(var + 1e-5) * g1_ref[...] + bt1_ref[...]
    r = jnp.maximum(hn, 0.0)
    o_ref[...] = _dotT(r, w2_ref[...]) + b2_ref[...] + nodes


def _layer(nodes, m0, m1, degb, ep, w1a, w1b, b1, g1, bt1, w2, b2):
    row = lambda i: (i, 0)
    full = lambda i: (0, 0)
    return pl.pallas_call(
        _layer_body,
        grid=(N // BR,),
        in_specs=[pl.BlockSpec((BR, D), row)] * 5
        + [pl.BlockSpec((D, D), full), pl.BlockSpec((D, D), full)]
        + [pl.BlockSpec((1, D), full)] * 3
        + [pl.BlockSpec((D, D), full), pl.BlockSpec((1, D), full)],
        out_specs=pl.BlockSpec((BR, D), row),
        out_shape=jax.ShapeDtypeStruct((N, D), _f32),
    )(nodes, m0, m1, degb, ep, w1a, w1b, b1.reshape(1, D), g1.reshape(1, D),
      bt1.reshape(1, D), w2, b2.reshape(1, D))


QB = 256


def _attn_body(q_ref, k_ref, v_ref, wo_ref, bo_ref, nodes_ref, o_ref, acc):
    h = pl.program_id(1)
    lane = lax.broadcasted_iota(jnp.int32, (1, D), 1)
    msk = (lane // HD == h).astype(_f32)
    km = k_ref[...] * msk
    s = _dotT(q_ref[...], km) * (1.0 / (HD ** 0.5))
    m = jnp.max(s, axis=1, keepdims=True)
    p = jnp.exp(s - m)
    z = jnp.sum(p, axis=1, keepdims=True)
    vm = v_ref[...] * msk
    ctx = lax.dot_general(p, vm, (((1,), (0,)), ((), ())),
                          preferred_element_type=_f32) / z

    @pl.when(h == 0)
    def _():
        acc[...] = ctx

    @pl.when(h > 0)
    def _():
        acc[...] += ctx

    @pl.when(h == H - 1)
    def _():
        o_ref[...] = _dotT(acc[...], wo_ref[...]) + bo_ref[...] + nodes_ref[...]


def _attn(q, k, v, w_out, b_out, nodes):
    qrow = lambda i, h: (i, 0)
    full = lambda i, h: (0, 0)
    return pl.pallas_call(
        _attn_body,
        grid=(N // QB, H),
        in_specs=[
            pl.BlockSpec((QB, D), qrow),
            pl.BlockSpec((N, D), full),
            pl.BlockSpec((N, D), full),
            pl.BlockSpec((D, D), full),
            pl.BlockSpec((1, D), full),
            pl.BlockSpec((QB, D), qrow),
        ],
        out_specs=pl.BlockSpec((QB, D), qrow),
        out_shape=jax.ShapeDtypeStruct((N, D), _f32),
        scratch_shapes=[pltpu.VMEM((QB, D), _f32)],
        compiler_params=pltpu.CompilerParams(
            dimension_semantics=("parallel", "arbitrary")),
    )(q, k, v, w_out, b_out.reshape(1, D), nodes)


def kernel(node_features, edge_index, edge_features, W_np, b_np, W_ep, b_ep,
           W1, b1, g1, bt1, W2, b2, W_in, b_in, W_out, b_out):
    src3 = edge_index[0].reshape(NW, NG, G)
    dst3 = edge_index[1].reshape(NW, NG, G)
    ef_p = edge_features.reshape(NW, NG, G // 8, D)
    zeros128 = jnp.zeros((RPS, D), _f32)
    wide_init = jnp.zeros((G, D), _f32).at[:, ED].set(1.0)

    efp = _prep_sc(dst3, ef_p, zeros128, wide_init)

    wpad = jnp.zeros((D, D), _f32).at[:, :ED].set(W_ep).at[:, ED].set(b_ep)
    cdeg = jnp.zeros((D, D), _f32).at[:, ED].set(1.0)
    ep, degb = _prep2(efp, wpad, cdeg)

    nodes = _proj(node_features, W_np, b_np, D)

    for l in range(L):
        parts = _msg_sc(nodes, src3, dst3, zeros128)
        nodes = _layer(nodes, parts[0], parts[1], degb, ep,
                       W1[l, :, :D], W1[l, :, D:], b1[l], g1[l], bt1[l],
                       W2[l], b2[l])

    qkv = _proj(nodes, W_in, b_in, 3 * D)
    q = qkv[:, :D]
    k = qkv[:, D:2 * D]
    v = qkv[:, 2 * D:]
    return _attn(q, k, v, W_out, b_out, nodes)


# speedup vs baseline: 1.4846x; 1.0386x over previous
import functools

import jax
import jax.numpy as jnp
from jax import lax
from jax.experimental import pallas as pl
from jax.experimental.pallas import tpu as pltpu
from jax.experimental.pallas import tpu_sc as plsc

N = 4096
E = 262144
D = 128
ED = 16
L = 3
H = 8
HD = D // H

NC = 2
NS = 16
NW = NC * NS
EPW = E // NW
G = 128
NG = EPW // G
RPS = N // NS

_f32 = jnp.float32


def _mesh():
    return plsc.VectorSubcoreMesh(core_axis_name="c", subcore_axis_name="s")


PB = 2


def _prep_sc_body(dst3, ef_p, zeros128, wide_init, out, acc, idx_d, *bs):
    pbufs, wides = bs[:PB], bs[PB:2 * PB]
    esems, ssems = bs[2 * PB:3 * PB], bs[3 * PB:]
    c = lax.axis_index("c")
    s = lax.axis_index("s")
    w = c * NS + s

    pltpu.sync_copy(zeros128, acc.at[pl.ds(s * RPS, RPS)])
    for p in range(PB):
        pltpu.sync_copy(wide_init, wides[p])
    pltpu.sync_copy(dst3.at[w], idx_d)
    plsc.subcore_barrier()

    for p in range(PB):
        pltpu.async_copy(ef_p.at[w, p], pbufs[p], esems[p])

    def body(q, carry):
        for p in range(PB):
            g = PB * q + p
            pltpu.make_async_copy(ef_p.at[w, g], pbufs[p], esems[p]).wait()

            @pl.when(g >= PB)
            def _():
                pltpu.make_async_copy(wides[p], acc.at[idx_d.at[g - PB]],
                                      ssems[p]).wait()

            for j in range(G):
                wides[p][j, pl.ds(0, ED)] = pbufs[p][j // 8,
                                                     pl.ds((j % 8) * ED, ED)]
            pltpu.async_copy(wides[p], acc.at[idx_d.at[g]], ssems[p],
                             add=True)

            @pl.when(g + PB < NG)
            def _():
                pltpu.async_copy(ef_p.at[w, g + PB], pbufs[p], esems[p])

        return carry

    lax.fori_loop(0, NG // PB, body, 0)
    for p in range(PB):
        pltpu.make_async_copy(wides[p], acc.at[idx_d.at[NG - PB + p]],
                              ssems[p]).wait()
    plsc.subcore_barrier()
    pltpu.sync_copy(acc.at[pl.ds(s * RPS, RPS)], out.at[c, pl.ds(s * RPS, RPS)])


def _prep_sc(dst3, ef_p, zeros128, wide_init):
    return pl.kernel(
        _prep_sc_body,
        out_type=jax.ShapeDtypeStruct((NC, N, D), _f32),
        mesh=_mesh(),
        scratch_types=[
            pltpu.VMEM_SHARED((N, D), _f32),
            pltpu.VMEM((NG, G), jnp.int32),
        ]
        + [pltpu.VMEM((G // 8, D), _f32)] * PB
        + [pltpu.VMEM((G, D), _f32)] * PB
        + [pltpu.SemaphoreType.DMA] * (2 * PB),
    )(dst3, ef_p, zeros128, wide_init)


NBUF = 4


def _msg_sc_body(nodes, src3, dst3, zeros128, out, acc, idx_s, idx_d, *bs):
    bufs, gsems, ssems = bs[:NBUF], bs[NBUF:2 * NBUF], bs[2 * NBUF:]
    c = lax.axis_index("c")
    s = lax.axis_index("s")
    w = c * NS + s

    pltpu.sync_copy(zeros128, acc.at[pl.ds(s * RPS, RPS)])
    pltpu.sync_copy(src3.at[w], idx_s)
    pltpu.sync_copy(dst3.at[w], idx_d)
    plsc.subcore_barrier()

    for p in range(NBUF):
        pltpu.async_copy(nodes.at[idx_s.at[p]], bufs[p], gsems[p])

    def body(q, carry):
        for p in range(NBUF):
            g = NBUF * q + p
            pltpu.make_async_copy(nodes.at[idx_s.at[g]], bufs[p], gsems[p]).wait()
            pltpu.async_copy(bufs[p], acc.at[idx_d.at[g]], ssems[p], add=True)

            @pl.when(g + NBUF < NG)
            def _():
                pltpu.make_async_copy(bufs[p], acc.at[idx_d.at[g]],
                                      ssems[p]).wait()
                pltpu.async_copy(nodes.at[idx_s.at[g + NBUF]], bufs[p],
                                 gsems[p])

        return carry

    lax.fori_loop(0, NG // NBUF, body, 0)
    for p in range(NBUF):
        pltpu.make_async_copy(bufs[p], acc.at[idx_d.at[NG - NBUF + p]],
                              ssems[p]).wait()
    plsc.subcore_barrier()
    pltpu.sync_copy(acc.at[pl.ds(s * RPS, RPS)], out.at[c, pl.ds(s * RPS, RPS)])


def _msg_sc(nodes, src3, dst3, zeros128):
    return pl.kernel(
        _msg_sc_body,
        out_type=jax.ShapeDtypeStruct((NC, N, D), _f32),
        mesh=_mesh(),
        scratch_types=[
            pltpu.VMEM_SHARED((N, D), _f32),
            pltpu.VMEM((NG, G), jnp.int32),
            pltpu.VMEM((NG, G), jnp.int32),
        ]
        + [pltpu.VMEM((G, D), _f32)] * NBUF
        + [pltpu.SemaphoreType.DMA] * (2 * NBUF),
    )(nodes, src3, dst3, zeros128)


def _dotT(x, w):
    return lax.dot_general(x, w, (((1,), (1,)), ((), ())),
                           preferred_element_type=_f32)


BR = 512


def _proj_body(x_ref, w_ref, b_ref, o_ref):
    o_ref[...] = _dotT(x_ref[...], w_ref[...]) + b_ref[...]


def _proj(x, w, b, out_cols):
    n = x.shape[0]
    return pl.pallas_call(
        _proj_body,
        grid=(n // BR,),
        in_specs=[
            pl.BlockSpec((BR, x.shape[1]), lambda i: (i, 0)),
            pl.BlockSpec(w.shape, lambda i: (0, 0)),
            pl.BlockSpec((1, out_cols), lambda i: (0, 0)),
        ],
        out_specs=pl.BlockSpec((BR, out_cols), lambda i: (i, 0)),
        out_shape=jax.ShapeDtypeStruct((n, out_cols), _f32),
    )(x, w, b.reshape(1, -1))


def _prep2_body(e0_ref, e1_ref, wpad_ref, cdeg_ref, ep_ref, degb_ref):
    sum_ef = e0_ref[...] + e1_ref[...]
    ep_ref[...] = _dotT(sum_ef, wpad_ref[...])
    degb_ref[...] = _dotT(sum_ef, cdeg_ref[...])


def _prep2(efp, wpad, cdeg):
    return pl.pallas_call(
        _prep2_body,
        grid=(N // BR,),
        in_specs=[
            pl.BlockSpec((BR, D), lambda i: (i, 0)),
            pl.BlockSpec((BR, D), lambda i: (i, 0)),
            pl.BlockSpec((D, D), lambda i: (0, 0)),
            pl.BlockSpec((D, D), lambda i: (0, 0)),
        ],
        out_specs=[
            pl.BlockSpec((BR, D), lambda i: (i, 0)),
            pl.BlockSpec((BR, D), lambda i: (i, 0)),
        ],
        out_shape=[
            jax.ShapeDtypeStruct((N, D), _f32),
            jax.ShapeDtypeStruct((N, D), _f32),
        ],
    )(efp[0], efp[1], wpad, cdeg)


def _layer_body(nodes_ref, m0_ref, m1_ref, degb_ref, ep_ref,
                w1a_ref, w1b_ref, b1_ref, g1_ref, bt1_ref, w2_ref, b2_ref,
                o_ref):
    nodes = nodes_ref[...]
    agg = m0_ref[...] + m1_ref[...] + degb_ref[...] * nodes + ep_ref[...]
    h = _dotT(nodes, w1a_ref[...]) + _dotT(agg, w1b_ref[...]) + b1_ref[...]
    mu = jnp.mean(h, axis=1, keepdims=True)
    dvar = h - mu
    var = jnp.mean(dvar * dvar, axis=1, keepdims=True)
    hn = dvar * lax.rsqrt(var + 1e-5) * g1_ref[...] + bt1_ref[...]
    r = jnp.maximum(hn, 0.0)
    o_ref[...] = _dotT(r, w2_ref[...]) + b2_ref[...] + nodes


def _layer(nodes, m0, m1, degb, ep, w1a, w1b, b1, g1, bt1, w2, b2):
    row = lambda i: (i, 0)
    full = lambda i: (0, 0)
    return pl.pallas_call(
        _layer_body,
        grid=(N // BR,),
        in_specs=[pl.BlockSpec((BR, D), row)] * 5
        + [pl.BlockSpec((D, D), full), pl.BlockSpec((D, D), full)]
        + [pl.BlockSpec((1, D), full)] * 3
        + [pl.BlockSpec((D, D), full), pl.BlockSpec((1, D), full)],
        out_specs=pl.BlockSpec((BR, D), row),
        out_shape=jax.ShapeDtypeStruct((N, D), _f32),
    )(nodes, m0, m1, degb, ep, w1a, w1b, b1.reshape(1, D), g1.reshape(1, D),
      bt1.reshape(1, D), w2, b2.reshape(1, D))


QB = 256


def _attn_body(q_ref, k_ref, v_ref, wo_ref, bo_ref, nodes_ref, o_ref, acc):
    h = pl.program_id(1)
    lane = lax.broadcasted_iota(jnp.int32, (1, D), 1)
    msk = (lane // HD == h).astype(_f32)
    km = k_ref[...] * msk
    s = _dotT(q_ref[...], km) * (1.0 / (HD ** 0.5))
    m = jnp.max(s, axis=1, keepdims=True)
    p = jnp.exp(s - m)
    z = jnp.sum(p, axis=1, keepdims=True)
    vm = v_ref[...] * msk
    ctx = lax.dot_general(p, vm, (((1,), (0,)), ((), ())),
                          preferred_element_type=_f32) / z

    @pl.when(h == 0)
    def _():
        acc[...] = ctx

    @pl.when(h > 0)
    def _():
        acc[...] += ctx

    @pl.when(h == H - 1)
    def _():
        o_ref[...] = _dotT(acc[...], wo_ref[...]) + bo_ref[...] + nodes_ref[...]


def _attn(q, k, v, w_out, b_out, nodes):
    qrow = lambda i, h: (i, 0)
    full = lambda i, h: (0, 0)
    return pl.pallas_call(
        _attn_body,
        grid=(N // QB, H),
        in_specs=[
            pl.BlockSpec((QB, D), qrow),
            pl.BlockSpec((N, D), full),
            pl.BlockSpec((N, D), full),
            pl.BlockSpec((D, D), full),
            pl.BlockSpec((1, D), full),
            pl.BlockSpec((QB, D), qrow),
        ],
        out_specs=pl.BlockSpec((QB, D), qrow),
        out_shape=jax.ShapeDtypeStruct((N, D), _f32),
        scratch_shapes=[pltpu.VMEM((QB, D), _f32)],
        compiler_params=pltpu.CompilerParams(
            dimension_semantics=("parallel", "arbitrary")),
    )(q, k, v, w_out, b_out.reshape(1, D), nodes)


def kernel(node_features, edge_index, edge_features, W_np, b_np, W_ep, b_ep,
           W1, b1, g1, bt1, W2, b2, W_in, b_in, W_out, b_out):
    src3 = edge_index[0].reshape(NW, NG, G)
    dst3 = edge_index[1].reshape(NW, NG, G)
    ef_p = edge_features.reshape(NW, NG, G // 8, D)
    zeros128 = jnp.zeros((RPS, D), _f32)
    wide_init = jnp.zeros((G, D), _f32).at[:, ED].set(1.0)

    efp = _prep_sc(dst3, ef_p, zeros128, wide_init)

    wpad = jnp.zeros((D, D), _f32).at[:, :ED].set(W_ep).at[:, ED].set(b_ep)
    cdeg = jnp.zeros((D, D), _f32).at[:, ED].set(1.0)
    ep, degb = _prep2(efp, wpad, cdeg)

    nodes = _proj(node_features, W_np, b_np, D)

    for l in range(L):
        parts = _msg_sc(nodes, src3, dst3, zeros128)
        nodes = _layer(nodes, parts[0], parts[1], degb, ep,
                       W1[l, :, :D], W1[l, :, D:], b1[l], g1[l], bt1[l],
                       W2[l], b2[l])

    qkv = _proj(nodes, W_in, b_in, 3 * D)
    q = qkv[:, :D]
    k = qkv[:, D:2 * D]
    v = qkv[:, 2 * D:]
    return _attn(q, k, v, W_out, b_out, nodes)
